# baseline (speedup 1.0000x reference)
import functools

import jax
import jax.numpy as jnp
from jax import lax
from jax.experimental import pallas as pl
from jax.experimental.pallas import tpu as pltpu
from jax.experimental.pallas import tpu_sc as plsc

_N = 3584
_KEEP = 7 * 512
_K = int(0.6 * _N)
_BR = 512
_STEPS = _N // _BR
_HB = _BR // 2
_NCH = _N // 16


def _sums_kernel(xa_ref, xb_ref, dw_ref, colabs_ref, dwc_ref):
    i = pl.program_id(0)
    xa = xa_ref[...]
    xb = xb_ref[...]
    rsa = jnp.sum(jnp.sum(xa, axis=0), axis=1, keepdims=True)
    rsb = jnp.sum(jnp.sum(xb, axis=0), axis=1, keepdims=True)
    dwc_ref[pl.ds(i * _BR, _HB), :] = rsa
    dwc_ref[pl.ds(i * _BR + _HB, _HB), :] = rsb
    part = jnp.sum(jnp.abs(xa), axis=1) + jnp.sum(jnp.abs(xb), axis=1)

    @pl.when(i == 0)
    def _():
        colabs_ref[0:1, 0:_KEEP] = jnp.zeros((1, _KEEP), jnp.float32)

    for j in range(7):
        colabs_ref[0:1, j * 512:(j + 1) * 512] = (
            colabs_ref[0:1, j * 512:(j + 1) * 512] + part[j:j + 1, :])

    @pl.when(i == _STEPS - 1)
    def _():
        dw_ref[...] = jnp.transpose(dwc_ref[...])


def _topk_sc(dw_hbm, colabs_hbm, out_hbm, dw_v, cl_v, bits_v, res_v):
    cid = lax.axis_index("c")
    sid = lax.axis_index("s")

    @pl.when((cid == 0) & (sid == 0))
    def _():
        pltpu.sync_copy(dw_hbm, dw_v)
        pltpu.sync_copy(colabs_hbm, cl_v)

        def absbody(i, carry):
            v = dw_v[pl.ds(i * 16, 16)]
            bits_v[pl.ds(i * 16, 16)] = plsc.bitcast(jnp.abs(v), jnp.int32)
            return carry

        lax.fori_loop(0, _NCH, absbody, 0)

        def bitbody(b, prefix):
            cand = prefix | lax.shift_left(jnp.int32(1), 30 - b)

            def cnt(i, acc):
                x = bits_v[pl.ds(i * 16, 16)]
                return acc + jnp.where(x < cand, 1, 0).astype(jnp.int32)

            c16 = lax.fori_loop(0, _NCH, cnt, jnp.zeros((16,), jnp.int32))
            c = jnp.sum(c16)
            return jnp.where(c >= _K, prefix, cand)

        thr = lax.fori_loop(0, 31, bitbody, jnp.int32(0))

        def cnt2(i, acc):
            x = bits_v[pl.ds(i * 16, 16)]
            return acc + jnp.where(x < thr, 1, 0).astype(jnp.int32)

        n_lt = jnp.sum(lax.fori_loop(0, _NCH, cnt2,
                                     jnp.zeros((16,), jnp.int32)))
        extra = _K - n_lt

        def fin(i, carry):
            tiecnt, acc = carry
            x = bits_v[pl.ds(i * 16, 16)]
            sel_lt = x < thr
            eq01 = jnp.where(x == thr, 1, 0).astype(jnp.int32)
            cum = plsc.cumsum(eq01)
            tie_ok = (eq01 > 0) & ((tiecnt + cum) <= extra)
            m = jnp.where(sel_lt | tie_ok, 1.0, 0.0)
            acc = acc + m * cl_v[pl.ds(i * 16, 16)]
            return (tiecnt + jnp.sum(eq01), acc)

        _, acc = lax.fori_loop(
            0, _NCH, fin,
            (jnp.int32(0), jnp.zeros((16,), jnp.float32)))
        res_v[...] = acc
        pltpu.sync_copy(res_v, out_hbm)


def kernel(ref_latents, iters):
    xt = jnp.transpose(ref_latents, (1, 0, 2))
    dw, colabs = pl.pallas_call(
        _sums_kernel,
        grid=(_STEPS,),
        in_specs=[pl.BlockSpec((7, _HB, 512), lambda i: (0, 2 * i, 0)),
                  pl.BlockSpec((7, _HB, 512), lambda i: (0, 2 * i + 1, 0))],
        out_specs=[pl.BlockSpec((1, _N), lambda i: (0, 0)),
                   pl.BlockSpec((1, _KEEP), lambda i: (0, 0))],
        out_shape=[jax.ShapeDtypeStruct((1, _N), jnp.float32),
                   jax.ShapeDtypeStruct((1, _KEEP), jnp.float32)],
        scratch_shapes=[pltpu.VMEM((_N, 1), jnp.float32)],
    )(xt, xt)

    mesh = plsc.VectorSubcoreMesh(core_axis_name="c", subcore_axis_name="s")
    topk = functools.partial(
        pl.kernel,
        out_type=jax.ShapeDtypeStruct((16,), jnp.float32),
        mesh=mesh,
        compiler_params=pltpu.CompilerParams(needs_layout_passes=False),
        scratch_types=[pltpu.VMEM((_N,), jnp.float32),
                       pltpu.VMEM((_N,), jnp.float32),
                       pltpu.VMEM((_N,), jnp.int32),
                       pltpu.VMEM((16,), jnp.float32)],
    )(_topk_sc)
    masked = topk(dw.reshape(_N), colabs.reshape(_KEEP))

    loss = jnp.sum(masked) / (_N * _KEEP)
    rw = jnp.maximum(0.0, (iters - 50) / (300 - 50))
    return rw * loss


# speedup vs baseline: 1.0326x; 1.0326x over previous
import functools

import jax
import jax.numpy as jnp
from jax import lax
from jax.experimental import pallas as pl
from jax.experimental.pallas import tpu as pltpu
from jax.experimental.pallas import tpu_sc as plsc

_N = 3584
_KEEP = 7 * 512
_K = int(0.6 * _N)
_BR = 512
_STEPS = _N // _BR
_HB = _BR // 2
_NCH = _N // 16


def _sums_kernel(xa_ref, xb_ref, dw_ref, colabs_ref, dwc_ref):
    i = pl.program_id(0)
    xa = xa_ref[...]
    xb = xb_ref[...]
    rsa = jnp.sum(jnp.sum(xa, axis=0), axis=1, keepdims=True)
    rsb = jnp.sum(jnp.sum(xb, axis=0), axis=1, keepdims=True)
    dwc_ref[pl.ds(i * _BR, _HB), :] = rsa
    dwc_ref[pl.ds(i * _BR + _HB, _HB), :] = rsb
    part = jnp.sum(jnp.abs(xa), axis=1) + jnp.sum(jnp.abs(xb), axis=1)

    @pl.when(i == 0)
    def _():
        colabs_ref[0:1, 0:_KEEP] = jnp.zeros((1, _KEEP), jnp.float32)

    for j in range(7):
        colabs_ref[0:1, j * 512:(j + 1) * 512] = (
            colabs_ref[0:1, j * 512:(j + 1) * 512] + part[j:j + 1, :])

    @pl.when(i == _STEPS - 1)
    def _():
        dw_ref[...] = jnp.transpose(dwc_ref[...])


def _topk_sc(dw_hbm, colabs_hbm, out_hbm, dw_v, cl_v, bits_v, res_v):
    cid = lax.axis_index("c")
    sid = lax.axis_index("s")

    @pl.when((cid == 0) & (sid == 0))
    def _():
        pltpu.sync_copy(dw_hbm, dw_v)
        pltpu.sync_copy(colabs_hbm, cl_v)

        def absbody(i, carry):
            v = dw_v[pl.ds(i * 16, 16)]
            bits_v[pl.ds(i * 16, 16)] = plsc.bitcast(jnp.abs(v), jnp.int32)
            return carry

        lax.fori_loop(0, _NCH, absbody, 0)

        def bitbody(b, carry):
            prefix, nlt = carry
            cand = prefix | lax.shift_left(jnp.int32(1), 30 - b)

            def cnt(i, acc):
                x = bits_v[pl.ds(i * 16, 16)]
                return acc + jnp.where(x < cand, 1, 0).astype(jnp.int32)

            c16 = lax.fori_loop(0, _NCH, cnt, jnp.zeros((16,), jnp.int32))
            c = jnp.sum(c16)
            keep = c >= _K
            return (jnp.where(keep, prefix, cand), jnp.where(keep, nlt, c))

        thr, n_lt = lax.fori_loop(0, 31, bitbody,
                                  (jnp.int32(0), jnp.int32(0)))
        extra = _K - n_lt

        def fin(i, carry):
            tiecnt, acc = carry
            x = bits_v[pl.ds(i * 16, 16)]
            sel_lt = x < thr
            eq01 = jnp.where(x == thr, 1, 0).astype(jnp.int32)
            cum = plsc.cumsum(eq01)
            tie_ok = (eq01 > 0) & ((tiecnt + cum) <= extra)
            m = jnp.where(sel_lt | tie_ok, 1.0, 0.0)
            acc = acc + m * cl_v[pl.ds(i * 16, 16)]
            return (tiecnt + jnp.sum(eq01), acc)

        _, acc = lax.fori_loop(
            0, _NCH, fin,
            (jnp.int32(0), jnp.zeros((16,), jnp.float32)))
        res_v[...] = jnp.broadcast_to(jnp.sum(acc), (16,))
        pltpu.sync_copy(res_v, out_hbm)


def kernel(ref_latents, iters):
    xt = jnp.transpose(ref_latents, (1, 0, 2))
    dw, colabs = pl.pallas_call(
        _sums_kernel,
        grid=(_STEPS,),
        in_specs=[pl.BlockSpec((7, _HB, 512), lambda i: (0, 2 * i, 0)),
                  pl.BlockSpec((7, _HB, 512), lambda i: (0, 2 * i + 1, 0))],
        out_specs=[pl.BlockSpec((1, _N), lambda i: (0, 0)),
                   pl.BlockSpec((1, _KEEP), lambda i: (0, 0))],
        out_shape=[jax.ShapeDtypeStruct((1, _N), jnp.float32),
                   jax.ShapeDtypeStruct((1, _KEEP), jnp.float32)],
        scratch_shapes=[pltpu.VMEM((_N, 1), jnp.float32)],
    )(xt, xt)

    mesh = plsc.VectorSubcoreMesh(core_axis_name="c", subcore_axis_name="s")
    topk = functools.partial(
        pl.kernel,
        out_type=jax.ShapeDtypeStruct((16,), jnp.float32),
        mesh=mesh,
        compiler_params=pltpu.CompilerParams(needs_layout_passes=False),
        scratch_types=[pltpu.VMEM((_N,), jnp.float32),
                       pltpu.VMEM((_N,), jnp.float32),
                       pltpu.VMEM((_N,), jnp.int32),
                       pltpu.VMEM((16,), jnp.float32)],
    )(_topk_sc)
    masked = topk(dw.reshape(_N), colabs.reshape(_KEEP))

    rw = jnp.maximum(0.0, (iters - 50) / (300 - 50))
    return rw * (masked[0] / (_N * _KEEP))


# speedup vs baseline: 1.3307x; 1.2887x over previous
import functools

import jax
import jax.numpy as jnp
from jax import lax
from jax.experimental import pallas as pl
from jax.experimental.pallas import tpu as pltpu
from jax.experimental.pallas import tpu_sc as plsc

_N = 3584
_KEEP = 7 * 512
_K = int(0.6 * _N)
_BR = 512
_STEPS = _N // _BR
_HB = _BR // 2
_NCH = _N // 16


def _sums_kernel(xa_ref, xb_ref, dw_ref, colabs_ref, dwc_ref):
    i = pl.program_id(0)
    xa = xa_ref[...]
    xb = xb_ref[...]
    rsa = jnp.sum(jnp.sum(xa, axis=0), axis=1, keepdims=True)
    rsb = jnp.sum(jnp.sum(xb, axis=0), axis=1, keepdims=True)
    dwc_ref[pl.ds(i * _BR, _HB), :] = rsa
    dwc_ref[pl.ds(i * _BR + _HB, _HB), :] = rsb
    part = jnp.sum(jnp.abs(xa), axis=1) + jnp.sum(jnp.abs(xb), axis=1)

    @pl.when(i == 0)
    def _():
        colabs_ref[0:1, 0:_KEEP] = jnp.zeros((1, _KEEP), jnp.float32)

    for j in range(7):
        colabs_ref[0:1, j * 512:(j + 1) * 512] = (
            colabs_ref[0:1, j * 512:(j + 1) * 512] + part[j:j + 1, :])

    @pl.when(i == _STEPS - 1)
    def _():
        dw_ref[...] = jnp.transpose(dwc_ref[...])


def _topk_sc(dw_hbm, colabs_hbm, out_hbm, dw_v, cl_v, bits_v, hist_v, res_v):
    cid = lax.axis_index("c")
    sid = lax.axis_index("s")

    @pl.when((cid == 0) & (sid == 0))
    def _():
        pltpu.sync_copy(dw_hbm, dw_v)
        pltpu.sync_copy(colabs_hbm, cl_v)

        def absbody(i, carry):
            v = dw_v[pl.ds(i * 16, 16)]
            bits_v[pl.ds(i * 16, 16)] = plsc.bitcast(jnp.abs(v), jnp.int32)
            return carry

        lax.fori_loop(0, _NCH, absbody, 0)

        one16 = jnp.ones((16,), jnp.int32)
        zero16 = jnp.zeros((16,), jnp.int32)
        lane = lax.iota(jnp.int32, 16)
        big = jnp.int32(1 << 30)

        def zero_hist(nchunks):
            def zb(i, c):
                hist_v[pl.ds(i * 16, 16)] = zero16
                return c
            lax.fori_loop(0, nchunks, zb, 0)

        def scan_find(nchunks, base):
            def sb(i, carry):
                cum, found = carry
                h = hist_v[pl.ds(i * 16, 16)]
                cumv = cum + plsc.cumsum(h)
                idx16 = lane + i * 16
                cand = jnp.min(jnp.where(base + cumv >= _K, idx16, big))
                return (cum + jnp.sum(h), jnp.minimum(found, cand))

            _, b = lax.fori_loop(0, nchunks, sb, (jnp.int32(0), big))

            def nb(i, acc):
                h = hist_v[pl.ds(i * 16, 16)]
                idx16 = lane + i * 16
                return acc + jnp.where(idx16 < b, h, 0)

            nbel = jnp.sum(lax.fori_loop(0, nchunks, nb,
                                         jnp.zeros((16,), jnp.int32)))
            return b, nbel

        zero_hist(32)

        def build_l1(i, c):
            b = bits_v[pl.ds(i * 16, 16)]
            plsc.addupdate_scatter(
                hist_v, [lax.shift_right_logical(b, 22)], one16)
            return c

        lax.fori_loop(0, _NCH, build_l1, 0)
        b9, nb1 = scan_find(32, jnp.int32(0))

        zero_hist(128)

        def build_l2(i, c):
            b = bits_v[pl.ds(i * 16, 16)]
            sel = lax.shift_right_logical(b, 22) == b9
            idx = jnp.bitwise_and(lax.shift_right_logical(b, 11), 2047)
            plsc.addupdate_scatter(hist_v, [idx], one16, mask=sel)
            return c

        lax.fori_loop(0, _NCH, build_l2, 0)
        b11, nb2 = scan_find(128, nb1)

        hi20 = jnp.bitwise_or(lax.shift_left(b9, 11), b11)
        zero_hist(128)

        def build_l3(i, c):
            b = bits_v[pl.ds(i * 16, 16)]
            sel = lax.shift_right_logical(b, 11) == hi20
            idx = jnp.bitwise_and(b, 2047)
            plsc.addupdate_scatter(hist_v, [idx], one16, mask=sel)
            return c

        lax.fori_loop(0, _NCH, build_l3, 0)
        b0, nb3 = scan_find(128, nb1 + nb2)

        thr = jnp.bitwise_or(lax.shift_left(hi20, 11), b0)
        n_lt = nb1 + nb2 + nb3
        extra = _K - n_lt

        def fin(i, carry):
            tiecnt, acc = carry
            x = bits_v[pl.ds(i * 16, 16)]
            sel_lt = x < thr
            eq01 = jnp.where(x == thr, 1, 0).astype(jnp.int32)
            cum = plsc.cumsum(eq01)
            tie_ok = (eq01 > 0) & ((tiecnt + cum) <= extra)
            m = jnp.where(sel_lt | tie_ok, 1.0, 0.0)
            acc = acc + m * cl_v[pl.ds(i * 16, 16)]
            return (tiecnt + jnp.sum(eq01), acc)

        _, acc = lax.fori_loop(
            0, _NCH, fin,
            (jnp.int32(0), jnp.zeros((16,), jnp.float32)))
        res_v[...] = jnp.broadcast_to(jnp.sum(acc), (16,))
        pltpu.sync_copy(res_v, out_hbm)


def kernel(ref_latents, iters):
    xt = jnp.transpose(ref_latents, (1, 0, 2))
    dw, colabs = pl.pallas_call(
        _sums_kernel,
        grid=(_STEPS,),
        in_specs=[pl.BlockSpec((7, _HB, 512), lambda i: (0, 2 * i, 0)),
                  pl.BlockSpec((7, _HB, 512), lambda i: (0, 2 * i + 1, 0))],
        out_specs=[pl.BlockSpec((1, _N), lambda i: (0, 0)),
                   pl.BlockSpec((1, _KEEP), lambda i: (0, 0))],
        out_shape=[jax.ShapeDtypeStruct((1, _N), jnp.float32),
                   jax.ShapeDtypeStruct((1, _KEEP), jnp.float32)],
        scratch_shapes=[pltpu.VMEM((_N, 1), jnp.float32)],
    )(xt, xt)

    mesh = plsc.VectorSubcoreMesh(core_axis_name="c", subcore_axis_name="s")
    topk = functools.partial(
        pl.kernel,
        out_type=jax.ShapeDtypeStruct((16,), jnp.float32),
        mesh=mesh,
        compiler_params=pltpu.CompilerParams(needs_layout_passes=False),
        scratch_types=[pltpu.VMEM((_N,), jnp.float32),
                       pltpu.VMEM((_N,), jnp.float32),
                       pltpu.VMEM((_N,), jnp.int32),
                       pltpu.VMEM((2048,), jnp.int32),
                       pltpu.VMEM((16,), jnp.float32)],
    )(_topk_sc)
    masked = topk(dw.reshape(_N), colabs.reshape(_KEEP))

    rw = jnp.maximum(0.0, (iters - 50) / (300 - 50))
    return rw * (masked[0] / (_N * _KEEP))


# speedup vs baseline: 1.3731x; 1.0318x over previous
import functools

import jax
import jax.numpy as jnp
from jax import lax
from jax.experimental import pallas as pl
from jax.experimental.pallas import tpu as pltpu
from jax.experimental.pallas import tpu_sc as plsc

_N = 3584
_KEEP = 7 * 512
_K = int(0.6 * _N)
_BR = 512
_STEPS = _N // _BR
_HB = _BR // 2
_NCH = _N // 16


def _sums_kernel(xa_ref, xb_ref, dw_ref, colabs_ref, dwc_ref):
    i = pl.program_id(0)
    xa = xa_ref[...]
    xb = xb_ref[...]
    rsa = jnp.sum(jnp.sum(xa, axis=0), axis=1, keepdims=True)
    rsb = jnp.sum(jnp.sum(xb, axis=0), axis=1, keepdims=True)
    dwc_ref[pl.ds(i * _BR, _HB), :] = rsa
    dwc_ref[pl.ds(i * _BR + _HB, _HB), :] = rsb
    part = jnp.sum(jnp.abs(xa), axis=1) + jnp.sum(jnp.abs(xb), axis=1)

    @pl.when(i == 0)
    def _():
        colabs_ref[0:1, 0:_KEEP] = jnp.zeros((1, _KEEP), jnp.float32)

    for j in range(7):
        colabs_ref[0:1, j * 512:(j + 1) * 512] = (
            colabs_ref[0:1, j * 512:(j + 1) * 512] + part[j:j + 1, :])

    @pl.when(i == _STEPS - 1)
    def _():
        dw_ref[...] = jnp.transpose(dwc_ref[...])


def _topk_sc(dw_hbm, colabs_hbm, out_hbm, dw_v, cl_v, bits_v, hist_v, res_v):
    cid = lax.axis_index("c")
    sid = lax.axis_index("s")

    @pl.when((cid == 0) & (sid == 0))
    def _():
        pltpu.sync_copy(dw_hbm, dw_v)
        pltpu.sync_copy(colabs_hbm, cl_v)

        one16 = jnp.ones((16,), jnp.int32)
        zero16 = jnp.zeros((16,), jnp.int32)
        lane = lax.iota(jnp.int32, 16)
        big = jnp.int32(1 << 30)

        def zero_hist(nchunks):
            def zb(i, c):
                hist_v[pl.ds(i * 16, 16)] = zero16
                return c
            lax.fori_loop(0, nchunks, zb, 0)

        def scan_find(nchunks, base):
            def sb(i, carry):
                cum, found = carry
                h = hist_v[pl.ds(i * 16, 16)]
                cumv = cum + plsc.cumsum(h)
                idx16 = lane + i * 16
                cand = jnp.min(jnp.where(base + cumv >= _K, idx16, big))
                return (cum + jnp.sum(h), jnp.minimum(found, cand))

            _, b = lax.fori_loop(0, nchunks, sb, (jnp.int32(0), big))

            def nb(i, acc):
                h = hist_v[pl.ds(i * 16, 16)]
                idx16 = lane + i * 16
                return acc + jnp.where(idx16 < b, h, 0)

            nbel = jnp.sum(lax.fori_loop(0, nchunks, nb,
                                         jnp.zeros((16,), jnp.int32)))
            return b, nbel

        zero_hist(32)

        def build_l1(i, c):
            for u in range(4):
                o = (i * 4 + u) * 16
                b = plsc.bitcast(jnp.abs(dw_v[pl.ds(o, 16)]), jnp.int32)
                bits_v[pl.ds(o, 16)] = b
                plsc.addupdate_scatter(
                    hist_v, [lax.shift_right_logical(b, 22)], one16)
            return c

        lax.fori_loop(0, _NCH // 4, build_l1, 0)
        b9, nb1 = scan_find(32, jnp.int32(0))

        zero_hist(128)

        def build_l2(i, c):
            for u in range(4):
                o = (i * 4 + u) * 16
                b = bits_v[pl.ds(o, 16)]
                sel = lax.shift_right_logical(b, 22) == b9
                idx = jnp.bitwise_and(lax.shift_right_logical(b, 11), 2047)
                plsc.addupdate_scatter(hist_v, [idx], one16, mask=sel)
            return c

        lax.fori_loop(0, _NCH // 4, build_l2, 0)
        b11, nb2 = scan_find(128, nb1)

        hi20 = jnp.bitwise_or(lax.shift_left(b9, 11), b11)
        zero_hist(128)

        def build_l3(i, c):
            for u in range(4):
                o = (i * 4 + u) * 16
                b = bits_v[pl.ds(o, 16)]
                sel = lax.shift_right_logical(b, 11) == hi20
                idx = jnp.bitwise_and(b, 2047)
                plsc.addupdate_scatter(hist_v, [idx], one16, mask=sel)
            return c

        lax.fori_loop(0, _NCH // 4, build_l3, 0)
        b0, nb3 = scan_find(128, nb1 + nb2)

        thr = jnp.bitwise_or(lax.shift_left(hi20, 11), b0)
        n_lt = nb1 + nb2 + nb3
        extra = _K - n_lt

        def fin(i, carry):
            tiecnt, acc = carry
            for u in range(4):
                o = (i * 4 + u) * 16
                x = bits_v[pl.ds(o, 16)]
                sel_lt = x < thr
                eq01 = jnp.where(x == thr, 1, 0).astype(jnp.int32)
                cum = plsc.cumsum(eq01)
                tie_ok = (eq01 > 0) & ((tiecnt + cum) <= extra)
                m = jnp.where(sel_lt | tie_ok, 1.0, 0.0)
                acc = acc + m * cl_v[pl.ds(o, 16)]
                tiecnt = tiecnt + jnp.sum(eq01)
            return (tiecnt, acc)

        _, acc = lax.fori_loop(
            0, _NCH // 4, fin,
            (jnp.int32(0), jnp.zeros((16,), jnp.float32)))
        res_v[...] = jnp.broadcast_to(jnp.sum(acc), (16,))
        pltpu.sync_copy(res_v, out_hbm)


def kernel(ref_latents, iters):
    xt = jnp.transpose(ref_latents, (1, 0, 2))
    dw, colabs = pl.pallas_call(
        _sums_kernel,
        grid=(_STEPS,),
        in_specs=[pl.BlockSpec((7, _HB, 512), lambda i: (0, 2 * i, 0)),
                  pl.BlockSpec((7, _HB, 512), lambda i: (0, 2 * i + 1, 0))],
        out_specs=[pl.BlockSpec((1, _N), lambda i: (0, 0)),
                   pl.BlockSpec((1, _KEEP), lambda i: (0, 0))],
        out_shape=[jax.ShapeDtypeStruct((1, _N), jnp.float32),
                   jax.ShapeDtypeStruct((1, _KEEP), jnp.float32)],
        scratch_shapes=[pltpu.VMEM((_N, 1), jnp.float32)],
    )(xt, xt)

    mesh = plsc.VectorSubcoreMesh(core_axis_name="c", subcore_axis_name="s")
    topk = functools.partial(
        pl.kernel,
        out_type=jax.ShapeDtypeStruct((16,), jnp.float32),
        mesh=mesh,
        compiler_params=pltpu.CompilerParams(needs_layout_passes=False),
        scratch_types=[pltpu.VMEM((_N,), jnp.float32),
                       pltpu.VMEM((_N,), jnp.float32),
                       pltpu.VMEM((_N,), jnp.int32),
                       pltpu.VMEM((2048,), jnp.int32),
                       pltpu.VMEM((16,), jnp.float32)],
    )(_topk_sc)
    masked = topk(dw.reshape(_N), colabs.reshape(_KEEP))

    rw = jnp.maximum(0.0, (iters - 50) / (300 - 50))
    return rw * (masked[0] / (_N * _KEEP))
